# baseline (speedup 1.0000x reference)
import functools

import jax
import jax.numpy as jnp
from jax import lax
from jax.experimental import pallas as pl
from jax.experimental.pallas import tpu as pltpu
from jax.experimental.pallas import tpu_sc as plsc

T, B, V, L = 32, 32, 50000, 400
TB = T * B
LANES = 16
NC = 2
NBUF = 2
VSPLIT = 24960


def _make_body(v0, w):
    def _dist_gen_body(vocab_hbm, attns_hbm, pg_hbm, src_hbm, out_hbm,
                       src_v, attn_v, pg_v, tmp_v, buf0, buf1,
                       in_sem0, in_sem1, out_sem0, out_sem1):
        wid = lax.axis_index("s") * NC + lax.axis_index("c")
        bufs = (buf0, buf1)
        in_sems = (in_sem0, in_sem1)
        out_sems = (out_sem0, out_sem1)

        for bi in range(NBUF):
            pltpu.async_copy(vocab_hbm.at[bi * B + wid], bufs[bi], in_sems[bi])
        pltpu.sync_copy(src_hbm.at[wid], src_v)
        pltpu.sync_copy(pg_hbm.at[wid], pg_v)
        pltpu.sync_copy(attns_hbm.at[wid], attn_v)

        def step(g, carry):
            for bi in range(NBUF):
                t = g * NBUF + bi
                r = t * B + wid
                pltpu.make_async_copy(
                    vocab_hbm.at[r], bufs[bi], in_sems[bi]).wait()

                tidx = jnp.zeros((LANES,), jnp.int32) + t
                p = plsc.load_gather(pg_v, [tidx])
                one_m_p = 1.0 - p

                for q in range(L // LANES):
                    sl = pl.ds(q * LANES, LANES)
                    s = src_v[sl] - v0
                    m = (s >= 0) & (s < w)
                    sc = jnp.where(m, s, 0)
                    tmp_v[sl] = plsc.load_gather(bufs[bi], [sc], mask=m)

                @plsc.parallel_loop(0, w // LANES, unroll=25)
                def scale_body(i):
                    sl = pl.ds(i * LANES, LANES)
                    bufs[bi][sl] = bufs[bi][sl] * p

                for q in range(L // LANES):
                    sl = pl.ds(q * LANES, LANES)
                    s = src_v[sl] - v0
                    m = (s >= 0) & (s < w)
                    sc = jnp.where(m, s, 0)
                    a = attn_v[pl.ds(t * L + q * LANES, LANES)]
                    val = one_m_p * a + p * tmp_v[sl]
                    plsc.store_scatter(bufs[bi], [sc], val, mask=m)

                pltpu.async_copy(bufs[bi], out_hbm.at[r], out_sems[bi])

            for bi in range(NBUF):
                t = g * NBUF + bi
                r = t * B + wid
                pltpu.make_async_copy(
                    bufs[bi], out_hbm.at[r], out_sems[bi]).wait()

                @pl.when(g < T // NBUF - 1)
                def _():
                    r2 = (t + NBUF) * B + wid
                    pltpu.async_copy(vocab_hbm.at[r2], bufs[bi], in_sems[bi])

            return carry

        lax.fori_loop(0, T // NBUF, step, 0)

    return _dist_gen_body


def _half_call(v0, w):
    mesh = plsc.VectorSubcoreMesh(core_axis_name="c", subcore_axis_name="s")
    return functools.partial(
        pl.kernel,
        out_type=jax.ShapeDtypeStruct((TB, w), jnp.float32),
        mesh=mesh,
        compiler_params=pltpu.CompilerParams(
            needs_layout_passes=False, use_tc_tiling_on_sc=True),
        scratch_types=[
            pltpu.VMEM((L,), jnp.int32),
            pltpu.VMEM((T * L,), jnp.float32),
            pltpu.VMEM((T,), jnp.float32),
            pltpu.VMEM((L,), jnp.float32),
            pltpu.VMEM((w,), jnp.float32),
            pltpu.VMEM((w,), jnp.float32),
            pltpu.SemaphoreType.DMA,
            pltpu.SemaphoreType.DMA,
            pltpu.SemaphoreType.DMA,
            pltpu.SemaphoreType.DMA,
        ],
    )(_make_body(v0, w))


@jax.jit
def _dist_gen(vocab_ds, attns_t, pg_bt, src_t):
    o0 = _half_call(0, VSPLIT)(
        vocab_ds[:, :VSPLIT], attns_t, pg_bt, src_t)
    o1 = _half_call(VSPLIT, V - VSPLIT)(
        vocab_ds[:, VSPLIT:], attns_t, pg_bt, src_t)
    return jnp.concatenate([o0, o1], axis=1)


def kernel(vocab_ds, attns, p_gens, sources, decoder_batch_len):
    del decoder_batch_len
    pg_bt = p_gens.reshape(T, B).T.reshape(B, T)
    src_t = sources.T.reshape(B, L)
    attns_t = attns.reshape(T, B, L).transpose(1, 0, 2).reshape(B, T * L)
    return _dist_gen(vocab_ds, attns_t, pg_bt, src_t)


# speedup vs baseline: 1.4312x; 1.4312x over previous
import functools

import jax
import jax.numpy as jnp
from jax import lax
from jax.experimental import pallas as pl
from jax.experimental.pallas import tpu as pltpu
from jax.experimental.pallas import tpu_sc as plsc

T, B, V, L = 32, 32, 50000, 400
TB = T * B
LANES = 16
NC = 2
NBUF = 2


def _dist_gen_body(vocab_hbm, attns_hbm, pg_hbm, src_hbm, out_hbm,
                   src_v, attn_v, pg_v, tmp_v, buf0, buf1,
                   in_sem0, in_sem1, out_sem0, out_sem1):
    wid = lax.axis_index("s") * NC + lax.axis_index("c")
    bufs = (buf0, buf1)
    in_sems = (in_sem0, in_sem1)
    out_sems = (out_sem0, out_sem1)

    for bi in range(NBUF):
        pltpu.async_copy(vocab_hbm.at[bi * B + wid], bufs[bi], in_sems[bi])
    pltpu.sync_copy(src_hbm.at[wid], src_v)
    pltpu.sync_copy(pg_hbm.at[wid], pg_v)
    pltpu.sync_copy(attns_hbm.at[wid], attn_v)

    def step(g, carry):
        for bi in range(NBUF):
            t = g * NBUF + bi
            r = t * B + wid
            pltpu.make_async_copy(vocab_hbm.at[r], bufs[bi], in_sems[bi]).wait()

            tidx = jnp.zeros((LANES,), jnp.int32) + t
            p = plsc.load_gather(pg_v, [tidx])
            one_m_p = 1.0 - p

            for q in range(L // LANES):
                sl = pl.ds(q * LANES, LANES)
                tmp_v[sl] = plsc.load_gather(bufs[bi], [src_v[sl]])

            @plsc.parallel_loop(0, V // LANES, unroll=25)
            def scale_body(i):
                sl = pl.ds(i * LANES, LANES)
                bufs[bi][sl] = bufs[bi][sl] * p

            for q in range(L // LANES):
                sl = pl.ds(q * LANES, LANES)
                a = attn_v[pl.ds(t * L + q * LANES, LANES)]
                val = one_m_p * a + p * tmp_v[sl]
                plsc.store_scatter(bufs[bi], [src_v[sl]], val)

            pltpu.async_copy(bufs[bi], out_hbm.at[r], out_sems[bi])

        for bi in range(NBUF):
            t = g * NBUF + bi
            r = t * B + wid
            pltpu.make_async_copy(bufs[bi], out_hbm.at[r], out_sems[bi]).wait()

            @pl.when(g < T // NBUF - 1)
            def _():
                r2 = (t + NBUF) * B + wid
                pltpu.async_copy(vocab_hbm.at[r2], bufs[bi], in_sems[bi])

        return carry

    lax.fori_loop(0, T // NBUF, step, 0)


@jax.jit
def _dist_gen(vocab_ds, attns_t, pg_bt, src_t):
    mesh = plsc.VectorSubcoreMesh(core_axis_name="c", subcore_axis_name="s")
    run = functools.partial(
        pl.kernel,
        out_type=jax.ShapeDtypeStruct((TB, V), jnp.float32),
        mesh=mesh,
        compiler_params=pltpu.CompilerParams(
            needs_layout_passes=False, use_tc_tiling_on_sc=True),
        scratch_types=[
            pltpu.VMEM((L,), jnp.int32),
            pltpu.VMEM((T * L,), jnp.float32),
            pltpu.VMEM((T,), jnp.float32),
            pltpu.VMEM((L,), jnp.float32),
            pltpu.VMEM((V,), jnp.float32),
            pltpu.VMEM((V,), jnp.float32),
            pltpu.SemaphoreType.DMA,
            pltpu.SemaphoreType.DMA,
            pltpu.SemaphoreType.DMA,
            pltpu.SemaphoreType.DMA,
        ],
    )(_dist_gen_body)
    return run(vocab_ds, attns_t, pg_bt, src_t)


def kernel(vocab_ds, attns, p_gens, sources, decoder_batch_len):
    del decoder_batch_len
    pg_bt = p_gens.reshape(T, B).T.reshape(B, T)
    src_t = sources.T.reshape(B, L)
    attns_t = attns.reshape(T, B, L).transpose(1, 0, 2).reshape(B, T * L)
    return _dist_gen(vocab_ds, attns_t, pg_bt, src_t)
